# baseline (speedup 1.0000x reference)
import functools

import jax
import jax.numpy as jnp
from jax import lax
from jax.experimental import pallas as pl
from jax.experimental.pallas import tpu as pltpu
from jax.experimental.pallas import tpu_sc as plsc

N = 10000
E = 160000
D = 128
K = 5000


_WSC = 1_250_000
_GWIN = _WSC + 128
_NPASS = N * N // (2 * _WSC)
_ECH = E // 16
_WCH = 25_000
_NCH = _WSC // _WCH


def _adj_body(src_hbm, dst_hbm, g_hbm, gwin, keyb, dstc, stage2d,
              ones_v, zeros_v, wbuf):
    c = lax.axis_index("c")
    s = lax.axis_index("s")
    lanes = lax.iota(jnp.int32, 16)

    def _zb(i, carry):
        wbuf[pl.ds(i * 16, 16)] = jnp.zeros((16,), jnp.float32)
        return carry
    lax.fori_loop(0, _WCH // 16, _zb, 0)
    for j in range(8):
        ones_v[pl.ds(j * 16, 16)] = jnp.ones((16,), jnp.float32)
        zeros_v[pl.ds(j * 16, 16)] = jnp.zeros((16,), jnp.float32)

    pltpu.sync_copy(src_hbm.at[pl.ds(s * _ECH, _ECH)], keyb)
    for j in range(5):
        pltpu.sync_copy(dst_hbm.at[pl.ds(s * _ECH + j * 2000, 2000)], dstc)

        def _key(i, carry):
            keyb[pl.ds(j * 2000 + i * 16, 16)] = (
                keyb[pl.ds(j * 2000 + i * 16, 16)] * N + dstc[pl.ds(i * 16, 16)])
            return carry
        lax.fori_loop(0, 2000 // 16, _key, 0)

    for j in range(4):
        @pl.when(s + 16 * j < _NCH)
        def _z():
            pltpu.sync_copy(wbuf, gwin.at[pl.ds((s + 16 * j) * _WCH, _WCH)])
    @pl.when(s == 0)
    def _zd():
        pltpu.sync_copy(wbuf.at[pl.ds(0, 128)], gwin.at[pl.ds(_WSC, 128)])
    plsc.subcore_barrier()

    def _pass(p, carry):
        base = p * (2 * _WSC) + c * _WSC

        def _scan(i, cnt):
            k = keyb[pl.ds(i * 16, 16)]
            t = k - base
            m = (t >= 0) & (t < _WSC)
            n_vec = plsc.all_reduce_population_count(m)
            _, tc = plsc.sort_key_val(jnp.where(m, 0, 1), t)
            pos = cnt + lanes
            plsc.store_scatter(stage2d, [pos >> 7, pos & 127], tc)
            return cnt + n_vec[0]
        cnt = lax.fori_loop(0, _ECH // 16, _scan, 0)

        for j in range(8):
            pos = cnt + j * 16 + lanes
            plsc.store_scatter(stage2d, [pos >> 7, pos & 127],
                               _WSC + j * 16 + lanes)
        nb = (cnt + 127) // 128

        def _sc(b, carry):
            pltpu.sync_copy(ones_v, gwin.at[stage2d.at[b]], add=True)
            return carry
        lax.fori_loop(0, nb, _sc, 0)
        plsc.subcore_barrier()

        for j in range(4):
            @pl.when(s + 16 * j < _NCH)
            def _wb():
                off = (s + 16 * j) * _WCH
                pltpu.sync_copy(gwin.at[pl.ds(off, _WCH)], wbuf)
                pltpu.sync_copy(wbuf, g_hbm.at[pl.ds(base + off, _WCH)])
        plsc.subcore_barrier()

        def _rz(b, carry):
            pltpu.sync_copy(zeros_v, gwin.at[stage2d.at[b]])
            return carry
        lax.fori_loop(0, nb, _rz, 0)
        plsc.subcore_barrier()
        return carry

    lax.fori_loop(0, _NPASS, _pass, 0)


@jax.jit
def _adj_counts(src, dst):
    mesh = plsc.VectorSubcoreMesh(core_axis_name="c", subcore_axis_name="s")
    f = pl.kernel(
        _adj_body,
        out_type=jax.ShapeDtypeStruct((N * N,), jnp.float32),
        mesh=mesh,
        scratch_types=[
            pltpu.VMEM_SHARED((_GWIN,), jnp.float32),
            pltpu.VMEM((_ECH,), jnp.int32),
            pltpu.VMEM((2000,), jnp.int32),
            pltpu.VMEM((80, 128), jnp.int32),
            pltpu.VMEM((128,), jnp.float32),
            pltpu.VMEM((128,), jnp.float32),
            pltpu.VMEM((_WCH,), jnp.float32),
        ],
        compiler_params=pltpu.CompilerParams(needs_layout_passes=False),
    )
    return f(src, dst)


_BM = 200


def _score_body(adj_ref, h_ref, wd_ref, bd_ref, wp_ref, bp_ref,
                g_ref, h1_ref, sc_ref, inv_ref):
    adj = adj_ref[...]
    deg = jnp.sum(adj, axis=1, keepdims=True)
    g = adj / (deg + 1e-8)
    g_ref[...] = g
    inv_ref[...] = 1.0 / (deg + 1e-8)
    t = jnp.dot(g, h_ref[...])
    h1 = jnp.maximum(jnp.dot(t, wd_ref[...]) + bd_ref[...], 0.0)
    h1_ref[...] = h1
    sgt = jnp.dot(h1, wp_ref[...]) + bp_ref[...]
    sc_ref[...] = 1.0 / (1.0 + jnp.exp(-sgt))


def _dense_scores(adj, h, W_down, b_down, W_pool, b_pool):
    grid = N // _BM
    return pl.pallas_call(
        _score_body,
        grid=(grid,),
        in_specs=[
            pl.BlockSpec((_BM, N), lambda i: (i, 0)),
            pl.BlockSpec((N, D), lambda i: (0, 0)),
            pl.BlockSpec((D, D), lambda i: (0, 0)),
            pl.BlockSpec((1, D), lambda i: (0, 0)),
            pl.BlockSpec((D, 1), lambda i: (0, 0)),
            pl.BlockSpec((1, 1), lambda i: (0, 0)),
        ],
        out_specs=[
            pl.BlockSpec((_BM, N), lambda i: (i, 0)),
            pl.BlockSpec((_BM, D), lambda i: (i, 0)),
            pl.BlockSpec((_BM, 1), lambda i: (i, 0)),
            pl.BlockSpec((_BM, 1), lambda i: (i, 0)),
        ],
        out_shape=[
            jax.ShapeDtypeStruct((N, N), jnp.float32),
            jax.ShapeDtypeStruct((N, D), jnp.float32),
            jax.ShapeDtypeStruct((N, 1), jnp.float32),
            jax.ShapeDtypeStruct((N, 1), jnp.float32),
        ],
    )(adj, h, W_down, b_down.reshape(1, D), W_pool, b_pool.reshape(1, 1))


def kernel(h, edge_index, W_down0, b_down0, W_pool0, b_pool0, W_bot, b_bot, W_up0, b_up0):
    src, dst = edge_index[0], edge_index[1]
    adj = _adj_counts(src, dst).reshape(N, N)
    g, h1, sc, inv = _dense_scores(adj, h, W_down0, b_down0, W_pool0, b_pool0)
    scores = sc.reshape(N)
    vals, idx = jax.lax.top_k(scores, K)
    h2 = h1[idx] * vals[:, None]
    g2 = (g[idx][:, idx])
    g2 = g2 / (g2.sum(axis=1, keepdims=True) + 1e-8)
    att = jax.nn.softmax((h2 @ h2.T) / jnp.sqrt(jnp.float32(D)), axis=-1)
    g2r = g2 + att
    g2r = g2r / (g2r.sum(axis=1, keepdims=True) + 1e-8)
    h3 = jax.nn.relu((g2r @ h2) @ W_bot + b_bot)
    h_up = jnp.zeros((N, D), jnp.float32).at[idx].set(h3)
    h_up = h_up + h1
    out = (g @ h_up) @ W_up0 + b_up0
    return (out, att, h2, g)
